# baseline (speedup 1.0000x reference)
import functools

import jax
import jax.numpy as jnp
from jax import lax
from jax.experimental import pallas as pl
from jax.experimental.pallas import tpu as pltpu
from jax.experimental.pallas import tpu_sc as plsc

NC = 2
NS = 16
L = 16
NW = NC * NS
D = 64
CH = 128


def _body(hs_hbm, rs_hbm, ts_hbm, ere_hbm, eim_hbm, rre_hbm, rim_hbm,
          out_hbm, hs_v, rs_v, ts_v, ehre_v, ehim_v, etre_v, etim_v,
          rre_v, rim_v, out_v, sem, *, rows_per_w):
    wid = lax.axis_index("s") * NC + lax.axis_index("c")
    rows0 = jnp.arange(L, dtype=jnp.int32)

    for c in range(rows_per_w // CH):
        base = wid * rows_per_w + c * CH
        pltpu.sync_copy(hs_hbm.at[pl.ds(base, CH)], hs_v)
        pltpu.sync_copy(rs_hbm.at[pl.ds(base, CH)], rs_v)
        pltpu.sync_copy(ts_hbm.at[pl.ds(base, CH)], ts_v)
        copies = [
            pltpu.async_copy(ere_hbm.at[hs_v], ehre_v, sem),
            pltpu.async_copy(eim_hbm.at[hs_v], ehim_v, sem),
            pltpu.async_copy(ere_hbm.at[ts_v], etre_v, sem),
            pltpu.async_copy(eim_hbm.at[ts_v], etim_v, sem),
            pltpu.async_copy(rre_hbm.at[rs_v], rre_v, sem),
            pltpu.async_copy(rim_hbm.at[rs_v], rim_v, sem),
        ]
        for cp in copies:
            cp.wait()

        def group_body(g, _):
            rows = g * L + rows0

            def d_body(d, acc):
                cols = jnp.full((L,), d, dtype=jnp.int32)
                ehre = plsc.load_gather(ehre_v, [rows, cols])
                ehim = plsc.load_gather(ehim_v, [rows, cols])
                etre = plsc.load_gather(etre_v, [rows, cols])
                etim = plsc.load_gather(etim_v, [rows, cols])
                rre = plsc.load_gather(rre_v, [rows, cols])
                rim = plsc.load_gather(rim_v, [rows, cols])
                return (acc + rre * (ehre * etre + ehim * etim)
                        + rim * (ehre * etim - ehim * etre))

            acc = lax.fori_loop(0, D, d_body, jnp.zeros((L,), jnp.float32))
            out_v[pl.ds(g * L, L)] = acc
            return 0

        lax.fori_loop(0, CH // L, group_body, 0)
        pltpu.sync_copy(out_v, out_hbm.at[pl.ds(base, CH)])


def kernel(hs, rs, ts, ent_re, ent_im, rel_re, rel_im):
    batch = hs.shape[0]
    rows_per_w = batch // NW
    mesh = plsc.VectorSubcoreMesh(core_axis_name="c", subcore_axis_name="s")
    k = pl.kernel(
        functools.partial(_body, rows_per_w=rows_per_w),
        out_type=jax.ShapeDtypeStruct((batch,), jnp.float32),
        mesh=mesh,
        compiler_params=pltpu.CompilerParams(needs_layout_passes=False, use_tc_tiling_on_sc=False),
        scratch_types=[
            pltpu.VMEM((CH,), jnp.int32),
            pltpu.VMEM((CH,), jnp.int32),
            pltpu.VMEM((CH,), jnp.int32),
            pltpu.VMEM((CH, D), jnp.float32),
            pltpu.VMEM((CH, D), jnp.float32),
            pltpu.VMEM((CH, D), jnp.float32),
            pltpu.VMEM((CH, D), jnp.float32),
            pltpu.VMEM((CH, D), jnp.float32),
            pltpu.VMEM((CH, D), jnp.float32),
            pltpu.VMEM((CH,), jnp.float32),
            pltpu.SemaphoreType.DMA,
        ],
    )
    return k(hs, rs, ts, ent_re, ent_im, rel_re, rel_im)


# speedup vs baseline: 1.7024x; 1.7024x over previous
import functools

import jax
import jax.numpy as jnp
from jax import lax
from jax.experimental import pallas as pl
from jax.experimental.pallas import tpu as pltpu
from jax.experimental.pallas import tpu_sc as plsc

NC = 2
NS = 16
L = 16
NW = NC * NS
D = 64
SUB = 8
CH = 16


def _body(hb_hbm, hsub_hbm, rs_hbm, tb_hbm, tsub_hbm, ere_hbm, eim_hbm,
          rel_hbm, out_hbm, hb_v, tb_v, rs_v, hsub_v, tsub_v,
          ehre_v, ehim_v, etre_v, etim_v, rel_v, out_v, sem, rsem,
          *, rows_per_w):
    wid = lax.axis_index("s") * NC + lax.axis_index("c")
    rows0 = jnp.arange(L, dtype=jnp.int32)

    def chunk_body(c, _):
        base = wid * rows_per_w + c * CH
        pltpu.sync_copy(hb_hbm.at[pl.ds(base, CH)], hb_v)
        pltpu.sync_copy(tb_hbm.at[pl.ds(base, CH)], tb_v)
        pltpu.sync_copy(rs_hbm.at[pl.ds(base, CH)], rs_v)
        pltpu.sync_copy(hsub_hbm.at[pl.ds(base, CH)], hsub_v)
        pltpu.sync_copy(tsub_hbm.at[pl.ds(base, CH)], tsub_v)
        rel_cp = pltpu.async_copy(rel_hbm.at[rs_v], rel_v, rsem)

        hb_vec = hb_v[...]
        tb_vec = tb_v[...]
        for i in range(CH):
            hb = hb_vec[i]
            tb = tb_vec[i]
            pltpu.async_copy(ere_hbm.at[hb], ehre_v.at[i], sem)
            pltpu.async_copy(eim_hbm.at[hb], ehim_v.at[i], sem)
            pltpu.async_copy(ere_hbm.at[tb], etre_v.at[i], sem)
            pltpu.async_copy(eim_hbm.at[tb], etim_v.at[i], sem)
        pltpu.make_async_copy(ere_hbm.at[0], ehre_v, sem).wait()
        pltpu.make_async_copy(eim_hbm.at[0], ehim_v, sem).wait()
        pltpu.make_async_copy(ere_hbm.at[0], etre_v, sem).wait()
        pltpu.make_async_copy(eim_hbm.at[0], etim_v, sem).wait()
        rel_cp.wait()

        def group_body(g, _):
            rows = g * L + rows0
            hsub = hsub_v[pl.ds(g * L, L)]
            tsub = tsub_v[pl.ds(g * L, L)]

            def d_body(d, acc):
                cols = jnp.full((L,), d, dtype=jnp.int32)
                ehre = plsc.load_gather(ehre_v, [rows, hsub, cols])
                ehim = plsc.load_gather(ehim_v, [rows, hsub, cols])
                etre = plsc.load_gather(etre_v, [rows, tsub, cols])
                etim = plsc.load_gather(etim_v, [rows, tsub, cols])
                rre = plsc.load_gather(rel_v, [rows, cols])
                rim = plsc.load_gather(rel_v, [rows, cols + D])
                return (acc + rre * (ehre * etre + ehim * etim)
                        + rim * (ehre * etim - ehim * etre))

            acc = lax.fori_loop(0, D, d_body, jnp.zeros((L,), jnp.float32))
            out_v[pl.ds(g * L, L)] = acc
            return 0

        lax.fori_loop(0, CH // L, group_body, 0)
        pltpu.sync_copy(out_v, out_hbm.at[pl.ds(base, CH)])
        return 0

    lax.fori_loop(0, rows_per_w // CH, chunk_body, 0)


def kernel(hs, rs, ts, ent_re, ent_im, rel_re, rel_im):
    batch = hs.shape[0]
    rows_per_w = batch // NW
    num_ent = ent_re.shape[0]
    ere3 = ent_re.reshape(num_ent // SUB, SUB, D)
    eim3 = ent_im.reshape(num_ent // SUB, SUB, D)
    rel = jnp.concatenate([rel_re, rel_im], axis=1)
    hb = lax.shift_right_logical(hs, 3)
    hsub = lax.bitwise_and(hs, 7)
    tb = lax.shift_right_logical(ts, 3)
    tsub = lax.bitwise_and(ts, 7)
    mesh = plsc.VectorSubcoreMesh(core_axis_name="c", subcore_axis_name="s")
    k = pl.kernel(
        functools.partial(_body, rows_per_w=rows_per_w),
        out_type=jax.ShapeDtypeStruct((batch,), jnp.float32),
        mesh=mesh,
        compiler_params=pltpu.CompilerParams(needs_layout_passes=False),
        scratch_types=[
            pltpu.VMEM((CH,), jnp.int32),
            pltpu.VMEM((CH,), jnp.int32),
            pltpu.VMEM((CH,), jnp.int32),
            pltpu.VMEM((CH,), jnp.int32),
            pltpu.VMEM((CH,), jnp.int32),
            pltpu.VMEM((CH, SUB, D), jnp.float32),
            pltpu.VMEM((CH, SUB, D), jnp.float32),
            pltpu.VMEM((CH, SUB, D), jnp.float32),
            pltpu.VMEM((CH, SUB, D), jnp.float32),
            pltpu.VMEM((CH, 2 * D), jnp.float32),
            pltpu.VMEM((CH,), jnp.float32),
            pltpu.SemaphoreType.DMA,
            pltpu.SemaphoreType.DMA,
        ],
    )
    return k(hb, hsub, rs, tb, tsub, ere3, eim3, rel)


# speedup vs baseline: 1.8838x; 1.1066x over previous
import functools

import jax
import jax.numpy as jnp
from jax import lax
from jax.experimental import pallas as pl
from jax.experimental.pallas import tpu as pltpu
from jax.experimental.pallas import tpu_sc as plsc

NC = 2
NS = 16
L = 16
NW = NC * NS
D = 64
SUB = 8
CH = 16


def _body(hb_hbm, hsub_hbm, rs_hbm, tb_hbm, tsub_hbm, ere_hbm, eim_hbm,
          rel_hbm, out_hbm, hb_v, tb_v, rs_v, hsub_v, tsub_v,
          ehre_v, ehim_v, etre_v, etim_v, rel_v, out_v, sem, rsem,
          *, rows_per_w):
    wid = lax.axis_index("s") * NC + lax.axis_index("c")
    rows0 = jnp.arange(L, dtype=jnp.int32)

    def chunk_body(c, _):
        base = wid * rows_per_w + c * CH
        pltpu.sync_copy(hb_hbm.at[pl.ds(base, CH)], hb_v)
        pltpu.sync_copy(tb_hbm.at[pl.ds(base, CH)], tb_v)
        pltpu.sync_copy(rs_hbm.at[pl.ds(base, CH)], rs_v)
        pltpu.sync_copy(hsub_hbm.at[pl.ds(base, CH)], hsub_v)
        pltpu.sync_copy(tsub_hbm.at[pl.ds(base, CH)], tsub_v)
        rel_cp = pltpu.async_copy(rel_hbm.at[rs_v], rel_v, rsem)

        hb_vec = hb_v[...]
        tb_vec = tb_v[...]
        hsub_vec = hsub_v[...]
        tsub_vec = tsub_v[...]
        for i in range(CH):
            hb = hb_vec[i]
            tb = tb_vec[i]
            hsb = hsub_vec[i]
            tsb = tsub_vec[i]
            pltpu.async_copy(ere_hbm.at[hb, hsb], ehre_v.at[i], sem)
            pltpu.async_copy(eim_hbm.at[hb, hsb], ehim_v.at[i], sem)
            pltpu.async_copy(ere_hbm.at[tb, tsb], etre_v.at[i], sem)
            pltpu.async_copy(eim_hbm.at[tb, tsb], etim_v.at[i], sem)
        pltpu.make_async_copy(ere_hbm.at[0], ehre_v, sem).wait()
        pltpu.make_async_copy(eim_hbm.at[0], ehim_v, sem).wait()
        pltpu.make_async_copy(ere_hbm.at[0], etre_v, sem).wait()
        pltpu.make_async_copy(eim_hbm.at[0], etim_v, sem).wait()
        rel_cp.wait()

        def group_body(g, _):
            rows = g * L + rows0

            def d_body(d, acc):
                cols = jnp.full((L,), d, dtype=jnp.int32)
                ehre = plsc.load_gather(ehre_v, [rows, cols])
                ehim = plsc.load_gather(ehim_v, [rows, cols])
                etre = plsc.load_gather(etre_v, [rows, cols])
                etim = plsc.load_gather(etim_v, [rows, cols])
                rre = plsc.load_gather(rel_v, [rows, cols])
                rim = plsc.load_gather(rel_v, [rows, cols + D])
                return (acc + rre * (ehre * etre + ehim * etim)
                        + rim * (ehre * etim - ehim * etre))

            acc = lax.fori_loop(0, D, d_body, jnp.zeros((L,), jnp.float32))
            out_v[pl.ds(g * L, L)] = acc
            return 0

        lax.fori_loop(0, CH // L, group_body, 0)
        pltpu.sync_copy(out_v, out_hbm.at[pl.ds(base, CH)])
        return 0

    lax.fori_loop(0, rows_per_w // CH, chunk_body, 0)


def kernel(hs, rs, ts, ent_re, ent_im, rel_re, rel_im):
    batch = hs.shape[0]
    rows_per_w = batch // NW
    num_ent = ent_re.shape[0]
    ere3 = ent_re.reshape(num_ent // SUB, SUB, D)
    eim3 = ent_im.reshape(num_ent // SUB, SUB, D)
    rel = jnp.concatenate([rel_re, rel_im], axis=1)
    hb = lax.shift_right_logical(hs, 3)
    hsub = lax.bitwise_and(hs, 7)
    tb = lax.shift_right_logical(ts, 3)
    tsub = lax.bitwise_and(ts, 7)
    mesh = plsc.VectorSubcoreMesh(core_axis_name="c", subcore_axis_name="s")
    k = pl.kernel(
        functools.partial(_body, rows_per_w=rows_per_w),
        out_type=jax.ShapeDtypeStruct((batch,), jnp.float32),
        mesh=mesh,
        compiler_params=pltpu.CompilerParams(needs_layout_passes=False),
        scratch_types=[
            pltpu.VMEM((CH,), jnp.int32),
            pltpu.VMEM((CH,), jnp.int32),
            pltpu.VMEM((CH,), jnp.int32),
            pltpu.VMEM((CH,), jnp.int32),
            pltpu.VMEM((CH,), jnp.int32),
            pltpu.VMEM((CH, D), jnp.float32),
            pltpu.VMEM((CH, D), jnp.float32),
            pltpu.VMEM((CH, D), jnp.float32),
            pltpu.VMEM((CH, D), jnp.float32),
            pltpu.VMEM((CH, 2 * D), jnp.float32),
            pltpu.VMEM((CH,), jnp.float32),
            pltpu.SemaphoreType.DMA,
            pltpu.SemaphoreType.DMA,
        ],
    )
    return k(hb, hsub, rs, tb, tsub, ere3, eim3, rel)


# speedup vs baseline: 2.1189x; 1.1248x over previous
import functools

import jax
import jax.numpy as jnp
from jax import lax
from jax.experimental import pallas as pl
from jax.experimental.pallas import tpu as pltpu
from jax.experimental.pallas import tpu_sc as plsc

NC = 2
NS = 16
L = 16
NW = NC * NS
D = 64
SUB = 8
CH = 128
NIDX = 5


def _body(pk_hbm, ere_hbm, eim_hbm, rel_hbm, out_hbm,
          idx_v, ehre_v, ehim_v, etre_v, etim_v, rel_v, out_v, sem, rsem,
          *, rows_per_w):
    wid = lax.axis_index("s") * NC + lax.axis_index("c")
    rows0 = jnp.arange(L, dtype=jnp.int32)
    n_chunks = rows_per_w // CH

    def chunk_body(c, _):
        chunk_id = wid * n_chunks + c
        base = chunk_id * CH
        pltpu.sync_copy(pk_hbm.at[chunk_id], idx_v)
        rel_cp = pltpu.async_copy(rel_hbm.at[idx_v.at[4]], rel_v, rsem)

        for g in range(CH // L):
            hb_vec = idx_v[0, pl.ds(g * L, L)]
            tb_vec = idx_v[1, pl.ds(g * L, L)]
            hsub_vec = idx_v[2, pl.ds(g * L, L)]
            tsub_vec = idx_v[3, pl.ds(g * L, L)]
            for i in range(L):
                r = g * L + i
                hb = hb_vec[i]
                tb = tb_vec[i]
                hsb = hsub_vec[i]
                tsb = tsub_vec[i]
                pltpu.async_copy(ere_hbm.at[hb, hsb], ehre_v.at[r], sem)
                pltpu.async_copy(eim_hbm.at[hb, hsb], ehim_v.at[r], sem)
                pltpu.async_copy(ere_hbm.at[tb, tsb], etre_v.at[r], sem)
                pltpu.async_copy(eim_hbm.at[tb, tsb], etim_v.at[r], sem)
        pltpu.make_async_copy(ere_hbm.at[0], ehre_v, sem).wait()
        pltpu.make_async_copy(eim_hbm.at[0], ehim_v, sem).wait()
        pltpu.make_async_copy(ere_hbm.at[0], etre_v, sem).wait()
        pltpu.make_async_copy(eim_hbm.at[0], etim_v, sem).wait()
        rel_cp.wait()

        def group_body(g, _):
            rows = g * L + rows0

            def d_body(d, acc):
                cols = jnp.full((L,), d, dtype=jnp.int32)
                ehre = plsc.load_gather(ehre_v, [rows, cols])
                ehim = plsc.load_gather(ehim_v, [rows, cols])
                etre = plsc.load_gather(etre_v, [rows, cols])
                etim = plsc.load_gather(etim_v, [rows, cols])
                rre = plsc.load_gather(rel_v, [rows, cols])
                rim = plsc.load_gather(rel_v, [rows, cols + D])
                return (acc + rre * (ehre * etre + ehim * etim)
                        + rim * (ehre * etim - ehim * etre))

            acc = lax.fori_loop(0, D, d_body, jnp.zeros((L,), jnp.float32))
            out_v[pl.ds(g * L, L)] = acc
            return 0

        lax.fori_loop(0, CH // L, group_body, 0)
        pltpu.sync_copy(out_v, out_hbm.at[pl.ds(base, CH)])
        return 0

    lax.fori_loop(0, n_chunks, chunk_body, 0)


def kernel(hs, rs, ts, ent_re, ent_im, rel_re, rel_im):
    batch = hs.shape[0]
    rows_per_w = batch // NW
    num_ent = ent_re.shape[0]
    ere3 = ent_re.reshape(num_ent // SUB, SUB, D)
    eim3 = ent_im.reshape(num_ent // SUB, SUB, D)
    rel = jnp.concatenate([rel_re, rel_im], axis=1)
    hb = lax.shift_right_logical(hs, 3)
    hsub = lax.bitwise_and(hs, 7)
    tb = lax.shift_right_logical(ts, 3)
    tsub = lax.bitwise_and(ts, 7)
    pk = jnp.stack([hb, tb, hsub, tsub, rs], axis=0)
    pk = pk.reshape(NIDX, batch // CH, CH).transpose(1, 0, 2)
    mesh = plsc.VectorSubcoreMesh(core_axis_name="c", subcore_axis_name="s")
    k = pl.kernel(
        functools.partial(_body, rows_per_w=rows_per_w),
        out_type=jax.ShapeDtypeStruct((batch,), jnp.float32),
        mesh=mesh,
        compiler_params=pltpu.CompilerParams(needs_layout_passes=False),
        scratch_types=[
            pltpu.VMEM((NIDX, CH), jnp.int32),
            pltpu.VMEM((CH, D), jnp.float32),
            pltpu.VMEM((CH, D), jnp.float32),
            pltpu.VMEM((CH, D), jnp.float32),
            pltpu.VMEM((CH, D), jnp.float32),
            pltpu.VMEM((CH, 2 * D), jnp.float32),
            pltpu.VMEM((CH,), jnp.float32),
            pltpu.SemaphoreType.DMA,
            pltpu.SemaphoreType.DMA,
        ],
    )
    return k(pk, ere3, eim3, rel)
